# baseline (speedup 1.0000x reference)
import functools

import jax
import jax.numpy as jnp
from jax import lax
from jax.experimental import pallas as pl
from jax.experimental.pallas import tpu as pltpu
from jax.experimental.pallas import tpu_sc as plsc

NUM_USERS = 100000
NUM_BOOKS = 1000000
EMB = 64
BATCH = 16384

_TAKE_DNUMS = lax.GatherDimensionNumbers(
    offset_dims=(), collapsed_slice_dims=(0,), start_index_map=(0,))


def _lane_permute(x, idx):
    return lax.gather(x, idx[:, None], _TAKE_DNUMS, (1,),
                      mode=lax.GatherScatterMode.PROMISE_IN_BOUNDS)


NC = 2
NS = 16
NW = NC * NS
ROWS_PER_W = BATCH // NW
CHUNK = 128
NCHUNK = ROWS_PER_W // CHUNK


def _sc_body(uid_hbm, bid_hbm, ut_hbm, bt_hbm, w_hbm, b_hbm, out_hbm,
             idx_u, idx_b, u_rows, b_rows, wv, bv, out_v, sem):
    wid = lax.axis_index("s") * NC + lax.axis_index("c")
    base = wid * ROWS_PER_W

    pltpu.sync_copy(uid_hbm.at[wid], idx_u)
    pltpu.sync_copy(bid_hbm.at[wid], idx_b)
    pltpu.sync_copy(w_hbm, wv)
    pltpu.sync_copy(b_hbm, bv)

    copies = []
    for j in range(NCHUNK):
        copies.append(pltpu.async_copy(ut_hbm.at[idx_u.at[j]], u_rows.at[j], sem))
        copies.append(pltpu.async_copy(bt_hbm.at[idx_b.at[j]], b_rows.at[j], sem))
    for c in copies:
        c.wait()

    w0 = wv[pl.ds(0, 16)]
    w1 = wv[pl.ds(16, 16)]
    w2 = wv[pl.ds(32, 16)]
    w3 = wv[pl.ds(48, 16)]
    w4 = wv[pl.ds(64, 16)]
    w5 = wv[pl.ds(80, 16)]
    w6 = wv[pl.ds(96, 16)]
    w7 = wv[pl.ds(112, 16)]
    bias_v = bv[pl.ds(0, 16)]

    lane = lax.iota(jnp.int32, 16)
    rots = [(lane + k) & 15 for k in (8, 4, 2, 1)]
    masks = [lane == r for r in range(16)]

    for j in range(NCHUNK):
        def grp_body(g, _, j=j):
            outv = bias_v
            for r in range(16):
                i = g * 16 + r
                t0 = u_rows[j, i, pl.ds(0, 16)] * w0 + u_rows[j, i, pl.ds(16, 16)] * w1
                t1 = u_rows[j, i, pl.ds(32, 16)] * w2 + u_rows[j, i, pl.ds(48, 16)] * w3
                t2 = b_rows[j, i, pl.ds(0, 16)] * w4 + b_rows[j, i, pl.ds(16, 16)] * w5
                t3 = b_rows[j, i, pl.ds(32, 16)] * w6 + b_rows[j, i, pl.ds(48, 16)] * w7
                s = (t0 + t1) + (t2 + t3)
                for rot in rots:
                    s = s + _lane_permute(s, rot)
                outv = jnp.where(masks[r], s, outv)
            out_v[pl.ds(j * CHUNK + g * 16, 16)] = outv
            return 0
        lax.fori_loop(0, CHUNK // 16, grp_body, 0)

    pltpu.sync_copy(out_v, out_hbm.at[pl.ds(base, ROWS_PER_W)])


@jax.jit
def kernel(user_ids, book_ids, user_table, book_table, W, b):
    uid3 = user_ids.reshape(NW, NCHUNK, CHUNK)
    bid3 = book_ids.reshape(NW, NCHUNK, CHUNK)
    w_flat = W.reshape(EMB * 2)
    b_vec = jnp.broadcast_to(b, (16,))

    mesh = plsc.VectorSubcoreMesh(core_axis_name="c", subcore_axis_name="s")
    fn = functools.partial(
        pl.kernel,
        mesh=mesh,
        compiler_params=pltpu.CompilerParams(use_tc_tiling_on_sc=False),
        out_type=jax.ShapeDtypeStruct((BATCH,), jnp.float32),
        scratch_types=[
            pltpu.VMEM((NCHUNK, CHUNK), jnp.int32),
            pltpu.VMEM((NCHUNK, CHUNK), jnp.int32),
            pltpu.VMEM((NCHUNK, CHUNK, EMB), jnp.float32),
            pltpu.VMEM((NCHUNK, CHUNK, EMB), jnp.float32),
            pltpu.VMEM((EMB * 2,), jnp.float32),
            pltpu.VMEM((16,), jnp.float32),
            pltpu.VMEM((ROWS_PER_W,), jnp.float32),
            pltpu.SemaphoreType.DMA,
        ],
    )(_sc_body)
    return fn(uid3, bid3, user_table, book_table, w_flat, b_vec)
